# baseline (speedup 1.0000x reference)
import functools

import jax
import jax.numpy as jnp
from jax import lax
from jax.experimental import pallas as pl
from jax.experimental.pallas import tpu as pltpu
from jax.experimental.pallas import tpu_sc as plsc

B = 16384
K = 32
L = 16
NC = 2
NS = 16
NW = NC * NS
BPW = B // NW
GROUPS = BPW // L

_mesh = plsc.VectorSubcoreMesh(core_axis_name="c", subcore_axis_name="s")


@functools.partial(
    pl.kernel,
    out_type=jax.ShapeDtypeStruct((B,), jnp.float32),
    mesh=_mesh,
    compiler_params=pltpu.CompilerParams(
        use_tc_tiling_on_sc=False, needs_layout_passes=False
    ),
    scratch_types=[
        pltpu.VMEM((BPW,), jnp.int32),
        pltpu.VMEM((BPW,), jnp.int32),
        pltpu.VMEM((BPW, K), jnp.float32),
        pltpu.VMEM((BPW, K), jnp.float32),
        pltpu.VMEM((BPW,), jnp.float32),
        pltpu.SemaphoreType.DMA,
        pltpu.SemaphoreType.DMA,
    ],
)
def _mf_score(u_hbm, i_hbm, ue_hbm, ie_hbm, out_hbm,
              uidx_v, iidx_v, ue_v, ie_v, out_v, usem, isem):
    wid = lax.axis_index("s") * NC + lax.axis_index("c")
    base = wid * BPW

    pltpu.sync_copy(u_hbm.at[pl.ds(base, BPW)], uidx_v)
    pltpu.sync_copy(i_hbm.at[pl.ds(base, BPW)], iidx_v)
    cu = pltpu.async_copy(ue_hbm.at[uidx_v], ue_v, usem)
    ci = pltpu.async_copy(ie_hbm.at[iidx_v], ie_v, isem)
    cu.wait()
    ci.wait()

    lanes = lax.iota(jnp.int32, L)

    def group(g, _):
        rows = g * L + lanes
        acc = jnp.zeros((L,), jnp.float32)
        for k in range(K):
            cols = jnp.full((L,), k, jnp.int32)
            uv = plsc.load_gather(ue_v, [rows, cols])
            iv = plsc.load_gather(ie_v, [rows, cols])
            acc = acc + uv * iv
        out_v[pl.ds(g * L, L)] = acc
        return 0

    lax.fori_loop(0, GROUPS, group, 0)

    pltpu.sync_copy(out_v, out_hbm.at[pl.ds(base, BPW)])


def kernel(u, i, user_emb, item_emb):
    return _mf_score(u, i, user_emb, item_emb)


# speedup vs baseline: 3.7638x; 3.7638x over previous
import functools

import jax
import jax.numpy as jnp
from jax import lax
from jax.experimental import pallas as pl
from jax.experimental.pallas import tpu as pltpu
from jax.experimental.pallas import tpu_sc as plsc

B = 16384
K = 32
L = 16
TW = 128
NC = 2
NS = 16
NW = NC * NS
BPW = B // NW
GROUPS = BPW // L
NBUF = 4

_mesh = plsc.VectorSubcoreMesh(core_axis_name="c", subcore_axis_name="s")


@functools.partial(
    pl.kernel,
    out_type=jax.ShapeDtypeStruct((B,), jnp.float32),
    mesh=_mesh,
    compiler_params=pltpu.CompilerParams(
        use_tc_tiling_on_sc=True, needs_layout_passes=False
    ),
    scratch_types=[
        pltpu.VMEM((BPW,), jnp.int32),
        pltpu.VMEM((BPW,), jnp.int32),
        pltpu.VMEM((NBUF, K, TW), jnp.float32),
        pltpu.VMEM((NBUF, K, TW), jnp.float32),
        pltpu.VMEM((BPW,), jnp.float32),
        pltpu.SemaphoreType.DMA,
        pltpu.SemaphoreType.DMA,
    ],
)
def _mf_score(u_hbm, i_hbm, uet_hbm, iet_hbm, out_hbm,
              uidx_v, iidx_v, ubuf, ibuf, out_v, usem, isem):
    wid = lax.axis_index("s") * NC + lax.axis_index("c")
    base = wid * BPW

    pltpu.sync_copy(u_hbm.at[pl.ds(base, BPW)], uidx_v)
    pltpu.sync_copy(i_hbm.at[pl.ds(base, BPW)], iidx_v)

    lanes = lax.iota(jnp.int32, L)
    rows_lo = lanes
    rows_hi = lanes + L

    def fetch(uvec, ivec, r):
        slot = r % NBUF
        cu = pl.multiple_of((uvec[r] // TW) * TW, TW)
        ci = pl.multiple_of((ivec[r] // TW) * TW, TW)
        pltpu.async_copy(uet_hbm.at[pl.ds(0, K), pl.ds(cu, TW)], ubuf.at[slot], usem)
        pltpu.async_copy(iet_hbm.at[pl.ds(0, K), pl.ds(ci, TW)], ibuf.at[slot], isem)

    def dot(uvec, ivec, r):
        slot = r % NBUF
        pltpu.make_async_copy(
            uet_hbm.at[pl.ds(0, K), pl.ds(0, TW)], ubuf.at[slot], usem).wait()
        pltpu.make_async_copy(
            iet_hbm.at[pl.ds(0, K), pl.ds(0, TW)], ibuf.at[slot], isem).wait()
        lu_v = jnp.full((L,), 0, jnp.int32) + (uvec[r] % TW)
        li_v = jnp.full((L,), 0, jnp.int32) + (ivec[r] % TW)
        u_lo = plsc.load_gather(ubuf.at[slot], [rows_lo, lu_v])
        u_hi = plsc.load_gather(ubuf.at[slot], [rows_hi, lu_v])
        i_lo = plsc.load_gather(ibuf.at[slot], [rows_lo, li_v])
        i_hi = plsc.load_gather(ibuf.at[slot], [rows_hi, li_v])
        return jnp.sum(u_lo * i_lo + u_hi * i_hi)

    def group(g, _):
        uvec = uidx_v[pl.ds(g * L, L)]
        ivec = iidx_v[pl.ds(g * L, L)]
        for q in range(NBUF):
            fetch(uvec, ivec, q)
        acc = jnp.zeros((L,), jnp.float32)
        for r in range(L):
            s = dot(uvec, ivec, r)
            if r + NBUF < L:
                fetch(uvec, ivec, r + NBUF)
            acc = jnp.where(lanes == r, s, acc)
        out_v[pl.ds(g * L, L)] = acc
        return 0

    lax.fori_loop(0, GROUPS, group, 0)
    pltpu.sync_copy(out_v, out_hbm.at[pl.ds(base, BPW)])


def kernel(u, i, user_emb, item_emb):
    return _mf_score(u, i, user_emb.T, item_emb.T)


# speedup vs baseline: 3.9516x; 1.0499x over previous
import functools

import jax
import jax.numpy as jnp
from jax import lax
from jax.experimental import pallas as pl
from jax.experimental.pallas import tpu as pltpu
from jax.experimental.pallas import tpu_sc as plsc

B = 16384
K = 32
L = 16
TW = 128
NC = 2
NS = 16
NW = NC * NS
BPW = B // NW
GROUPS = BPW // L
NBUF = 4

_mesh = plsc.VectorSubcoreMesh(core_axis_name="c", subcore_axis_name="s")


@functools.partial(
    pl.kernel,
    out_type=jax.ShapeDtypeStruct((B,), jnp.float32),
    mesh=_mesh,
    compiler_params=pltpu.CompilerParams(
        use_tc_tiling_on_sc=True, needs_layout_passes=False
    ),
    scratch_types=[
        pltpu.VMEM((BPW,), jnp.int32),
        pltpu.VMEM((BPW,), jnp.int32),
        pltpu.VMEM((NBUF, K, TW), jnp.float32),
        pltpu.VMEM((NBUF, K, TW), jnp.float32),
        pltpu.VMEM((BPW,), jnp.float32),
        pltpu.SemaphoreType.DMA,
        pltpu.SemaphoreType.DMA,
    ],
)
def _mf_score(u_hbm, i_hbm, uet_hbm, iet_hbm, out_hbm,
              uidx_v, iidx_v, ubuf, ibuf, out_v, usem, isem):
    wid = lax.axis_index("s") * NC + lax.axis_index("c")
    base = wid * BPW

    pltpu.sync_copy(u_hbm.at[pl.ds(base, BPW)], uidx_v)
    pltpu.sync_copy(i_hbm.at[pl.ds(base, BPW)], iidx_v)

    lanes = lax.iota(jnp.int32, L)
    rows_lo = lanes
    rows_hi = lanes + L

    def fetch(uvec, ivec, r):
        slot = r % NBUF
        cu = pl.multiple_of((uvec[r] // TW) * TW, TW)
        ci = pl.multiple_of((ivec[r] // TW) * TW, TW)
        pltpu.async_copy(uet_hbm.at[pl.ds(0, K), pl.ds(cu, TW)], ubuf.at[slot], usem)
        pltpu.async_copy(iet_hbm.at[pl.ds(0, K), pl.ds(ci, TW)], ibuf.at[slot], isem)

    def dot(uvec, ivec, r):
        slot = r % NBUF
        pltpu.make_async_copy(
            uet_hbm.at[pl.ds(0, K), pl.ds(0, TW)], ubuf.at[slot], usem).wait()
        pltpu.make_async_copy(
            iet_hbm.at[pl.ds(0, K), pl.ds(0, TW)], ibuf.at[slot], isem).wait()
        lu_v = jnp.full((L,), 0, jnp.int32) + (uvec[r] % TW)
        li_v = jnp.full((L,), 0, jnp.int32) + (ivec[r] % TW)
        u_lo = plsc.load_gather(ubuf.at[slot], [rows_lo, lu_v])
        u_hi = plsc.load_gather(ubuf.at[slot], [rows_hi, lu_v])
        i_lo = plsc.load_gather(ibuf.at[slot], [rows_lo, li_v])
        i_hi = plsc.load_gather(ibuf.at[slot], [rows_hi, li_v])
        return jnp.sum(u_lo * i_lo + u_hi * i_hi)

    def group(g, _):
        uvec = uidx_v[pl.ds(g * L, L)]
        ivec = iidx_v[pl.ds(g * L, L)]
        gn = jnp.minimum(g + 1, GROUPS - 1) * L
        uvn = uidx_v[pl.ds(gn, L)]
        ivn = iidx_v[pl.ds(gn, L)]
        acc = jnp.zeros((L,), jnp.float32)
        for r in range(L):
            s = dot(uvec, ivec, r)
            if r + NBUF < L:
                fetch(uvec, ivec, r + NBUF)
            else:
                @pl.when(g < GROUPS - 1)
                def _():
                    fetch(uvn, ivn, r + NBUF - L)
            acc = jnp.where(lanes == r, s, acc)
        out_v[pl.ds(g * L, L)] = acc
        return 0

    uv0 = uidx_v[pl.ds(0, L)]
    iv0 = iidx_v[pl.ds(0, L)]
    for q in range(NBUF):
        fetch(uv0, iv0, q)
    lax.fori_loop(0, GROUPS, group, 0)
    pltpu.sync_copy(out_v, out_hbm.at[pl.ds(base, BPW)])


def kernel(u, i, user_emb, item_emb):
    return _mf_score(u, i, user_emb.T, item_emb.T)


# speedup vs baseline: 4.5443x; 1.1500x over previous
import functools

import jax
import jax.numpy as jnp
from jax import lax
from jax.experimental import pallas as pl
from jax.experimental.pallas import tpu as pltpu
from jax.experimental.pallas import tpu_sc as plsc

B = 16384
K = 32
L = 16
TW = 128
NC = 2
NS = 16
NW = NC * NS
BPW = B // NW
GROUPS = BPW // L
NBUF = 8

_mesh = plsc.VectorSubcoreMesh(core_axis_name="c", subcore_axis_name="s")


@functools.partial(
    pl.kernel,
    out_type=jax.ShapeDtypeStruct((B,), jnp.float32),
    mesh=_mesh,
    compiler_params=pltpu.CompilerParams(
        use_tc_tiling_on_sc=True, needs_layout_passes=False
    ),
    scratch_types=[
        pltpu.VMEM((BPW,), jnp.int32),
        pltpu.VMEM((BPW,), jnp.int32),
        pltpu.VMEM((NBUF, K, TW), jnp.float32),
        pltpu.VMEM((NBUF, K, TW), jnp.float32),
        pltpu.VMEM((BPW,), jnp.float32),
        pltpu.SemaphoreType.DMA,
        pltpu.SemaphoreType.DMA,
    ],
)
def _mf_score(u_hbm, i_hbm, uet_hbm, iet_hbm, out_hbm,
              uidx_v, iidx_v, ubuf, ibuf, out_v, usem, isem):
    wid = lax.axis_index("s") * NC + lax.axis_index("c")
    base = wid * BPW

    pltpu.sync_copy(u_hbm.at[pl.ds(base, BPW)], uidx_v)
    pltpu.sync_copy(i_hbm.at[pl.ds(base, BPW)], iidx_v)

    lanes = lax.iota(jnp.int32, L)
    rows_lo = lanes
    rows_hi = lanes + L

    def fetch(uvec, ivec, r):
        slot = r % NBUF
        cu = pl.multiple_of((uvec[r] // TW) * TW, TW)
        ci = pl.multiple_of((ivec[r] // TW) * TW, TW)
        pltpu.async_copy(uet_hbm.at[pl.ds(0, K), pl.ds(cu, TW)], ubuf.at[slot], usem)
        pltpu.async_copy(iet_hbm.at[pl.ds(0, K), pl.ds(ci, TW)], ibuf.at[slot], isem)

    def dot(uvec, ivec, r):
        slot = r % NBUF
        pltpu.make_async_copy(
            uet_hbm.at[pl.ds(0, K), pl.ds(0, TW)], ubuf.at[slot], usem).wait()
        pltpu.make_async_copy(
            iet_hbm.at[pl.ds(0, K), pl.ds(0, TW)], ibuf.at[slot], isem).wait()
        lu_v = jnp.full((L,), 0, jnp.int32) + (uvec[r] % TW)
        li_v = jnp.full((L,), 0, jnp.int32) + (ivec[r] % TW)
        u_lo = plsc.load_gather(ubuf.at[slot], [rows_lo, lu_v])
        u_hi = plsc.load_gather(ubuf.at[slot], [rows_hi, lu_v])
        i_lo = plsc.load_gather(ibuf.at[slot], [rows_lo, li_v])
        i_hi = plsc.load_gather(ibuf.at[slot], [rows_hi, li_v])
        return jnp.sum(u_lo * i_lo + u_hi * i_hi)

    def group(g, _):
        uvec = uidx_v[pl.ds(g * L, L)]
        ivec = iidx_v[pl.ds(g * L, L)]
        gn = jnp.minimum(g + 1, GROUPS - 1) * L
        uvn = uidx_v[pl.ds(gn, L)]
        ivn = iidx_v[pl.ds(gn, L)]
        acc = jnp.zeros((L,), jnp.float32)
        for r in range(L):
            s = dot(uvec, ivec, r)
            if r + NBUF < L:
                fetch(uvec, ivec, r + NBUF)
            else:
                @pl.when(g < GROUPS - 1)
                def _():
                    fetch(uvn, ivn, r + NBUF - L)
            acc = jnp.where(lanes == r, s, acc)
        out_v[pl.ds(g * L, L)] = acc
        return 0

    uv0 = uidx_v[pl.ds(0, L)]
    iv0 = iidx_v[pl.ds(0, L)]
    for q in range(NBUF):
        fetch(uv0, iv0, q)
    lax.fori_loop(0, GROUPS, group, 0)
    pltpu.sync_copy(out_v, out_hbm.at[pl.ds(base, BPW)])


def kernel(u, i, user_emb, item_emb):
    return _mf_score(u, i, user_emb.T, item_emb.T)
